# Initial kernel scaffold; baseline (speedup 1.0000x reference)
import jax
import jax.numpy as jnp
from jax.experimental import pallas as pl


def kernel(input_ids, table, pos_embed):
    raise NotImplementedError("write your pallas kernel here")


# speedup vs baseline: 1.8451x; 1.8451x over previous
import functools

import jax
import jax.numpy as jnp
from jax import lax
from jax.experimental import pallas as pl
from jax.experimental.pallas import tpu as pltpu
from jax.experimental.pallas import tpu_sc as plsc

VOCAB = 100000
SEQ = 200
DIM = 128
BATCH = 4096
TOT = BATCH * SEQ
NC = 2
NS = 16
NW = NC * NS
PER_W = TOT // NW
K = 128
NCH = PER_W // K
LANES = 16

_mesh = plsc.VectorSubcoreMesh(core_axis_name="c", subcore_axis_name="s")


@functools.partial(
    pl.kernel,
    mesh=_mesh,
    out_type=jax.ShapeDtypeStruct((TOT, DIM), jnp.float32),
    scratch_types=[
        pltpu.VMEM((K,), jnp.int32),
        pltpu.VMEM((K, DIM), jnp.float32),
        pltpu.VMEM((2 * SEQ * DIM,), jnp.float32),
        pltpu.SemaphoreType.DMA,
    ],
)
def _emb_kernel(ids_hbm, table_hbm, pos2_hbm, out_hbm, idx_v, buf, pos_v, sem):
    wid = lax.axis_index("s") * NC + lax.axis_index("c")
    base = wid * PER_W
    pltpu.sync_copy(pos2_hbm, pos_v)

    def chunk_body(c, _):
        gbase = base + c * K
        pltpu.sync_copy(ids_hbm.at[pl.ds(gbase, K)], idx_v)
        pltpu.async_copy(table_hbm.at[idx_v], buf, sem).wait()
        poff = lax.rem(c * K, SEQ) * DIM

        def row_body(r, off):
            for j in range(DIM // LANES):
                sl = pl.ds(j * LANES, LANES)
                buf[r, sl] = buf[r, sl] + pos_v[pl.ds(off + j * LANES, LANES)]
            return off + DIM

        lax.fori_loop(0, K, row_body, poff)
        pltpu.sync_copy(buf, out_hbm.at[pl.ds(gbase, K)])
        return 0

    lax.fori_loop(0, NCH, chunk_body, 0)


def kernel(input_ids, table, pos_embed):
    ids_flat = input_ids.reshape(TOT)
    pos_flat = pos_embed.reshape(SEQ * DIM)
    pos2 = jnp.concatenate([pos_flat, pos_flat])
    out = _emb_kernel(ids_flat, table, pos2)
    return out.reshape(BATCH, SEQ, DIM)


# speedup vs baseline: 3.1596x; 1.7124x over previous
import functools

import jax
import jax.numpy as jnp
from jax import lax
from jax.experimental import pallas as pl
from jax.experimental.pallas import tpu as pltpu
from jax.experimental.pallas import tpu_sc as plsc

VOCAB = 100000
SEQ = 200
DIM = 128
BATCH = 4096
TOT = BATCH * SEQ
NC = 2
NS = 16
NW = NC * NS
PER_W = TOT // NW
K = 128
NCH = PER_W // K
LANES = 16
NBUF = 4
LEAD = 2
GROUPS = NCH // NBUF
RPI = 4

_mesh = plsc.VectorSubcoreMesh(core_axis_name="c", subcore_axis_name="s")


@functools.partial(
    pl.kernel,
    mesh=_mesh,
    out_type=jax.ShapeDtypeStruct((TOT, DIM), jnp.float32),
    scratch_types=[
        pltpu.VMEM((NBUF, K), jnp.int32),
        pltpu.VMEM((NBUF, K, DIM), jnp.float32),
        pltpu.VMEM((2 * SEQ * DIM,), jnp.float32),
    ] + [pltpu.SemaphoreType.DMA] * (2 * NBUF),
)
def _emb_kernel(ids_hbm, table_hbm, pos2_hbm, out_hbm, idx_v, buf, pos_v, *sems):
    sg = sems[:NBUF]
    so = sems[NBUF:]
    wid = lax.axis_index("s") * NC + lax.axis_index("c")
    base = wid * PER_W
    pltpu.sync_copy(pos2_hbm, pos_v)

    def start_gather(pc, b):
        pltpu.sync_copy(ids_hbm.at[pl.ds(base + pc * K, K)], idx_v.at[b])
        pltpu.async_copy(table_hbm.at[idx_v.at[b]], buf.at[b], sg[b])

    def wait_gather(b):
        pltpu.make_async_copy(table_hbm.at[idx_v.at[b]], buf.at[b], sg[b]).wait()

    def start_out(cc, b):
        pltpu.async_copy(buf.at[b], out_hbm.at[pl.ds(base + cc * K, K)], so[b])

    def wait_out(cc, b):
        pltpu.make_async_copy(
            buf.at[b], out_hbm.at[pl.ds(base + cc * K, K)], so[b]
        ).wait()

    def add_pos(cc, b):
        poff = lax.rem(cc * K, SEQ) * DIM

        def rows(r0, off):
            for rr in range(RPI):
                for j in range(DIM // LANES):
                    v = pos_v[pl.ds(off + rr * DIM + j * LANES, LANES)]
                    plsc.addupdate(
                        buf.at[b, r0 * RPI + rr, pl.ds(j * LANES, LANES)], v
                    )
            return off + RPI * DIM

        lax.fori_loop(0, K // RPI, rows, poff)

    def step(cc, b, wait_prev_out, start_next):
        wait_gather(b)
        add_pos(cc, b)
        start_out(cc, b)
        b2 = (b + LEAD) % NBUF
        if wait_prev_out:
            wait_out(cc - LEAD, b2)
        if start_next:
            start_gather(cc + LEAD, b2)

    for b in range(LEAD):
        start_gather(b, b)
    for b in range(NBUF):
        step(b, b, wait_prev_out=(b >= LEAD), start_next=True)

    def group(g, _):
        c0 = g * NBUF
        for b in range(NBUF):
            step(c0 + b, b, wait_prev_out=True, start_next=True)
        return 0

    lax.fori_loop(1, GROUPS - 1, group, 0)

    c0 = (GROUPS - 1) * NBUF
    for b in range(NBUF):
        step(c0 + b, b, wait_prev_out=True, start_next=(b < LEAD))
    for b in range(LEAD, NBUF):
        wait_out(c0 + b, b)


def kernel(input_ids, table, pos_embed):
    ids_flat = input_ids.reshape(TOT)
    pos_flat = pos_embed.reshape(SEQ * DIM)
    pos2 = jnp.concatenate([pos_flat, pos_flat])
    out = _emb_kernel(ids_flat, table, pos2)
    return out.reshape(BATCH, SEQ, DIM)


# speedup vs baseline: 3.6416x; 1.1526x over previous
import functools

import jax
import jax.numpy as jnp
from jax import lax
from jax.experimental import pallas as pl
from jax.experimental.pallas import tpu as pltpu
from jax.experimental.pallas import tpu_sc as plsc

VOCAB = 100000
SEQ = 200
DIM = 128
BATCH = 4096
TOT = BATCH * SEQ
NC = 2
NS = 16
NW = NC * NS
PER_W = TOT // NW
K = 128
NCH = PER_W // K
LANES = 16
NBUF = 4
LEAD = 2
BLK = 8
NBLK = NCH // BLK
RPI = 4

_mesh = plsc.VectorSubcoreMesh(core_axis_name="c", subcore_axis_name="s")


@functools.partial(
    pl.kernel,
    mesh=_mesh,
    out_type=jax.ShapeDtypeStruct((TOT, DIM), jnp.float32),
    scratch_types=[
        pltpu.VMEM((2, BLK, K), jnp.int32),
        pltpu.VMEM((NBUF, K, DIM), jnp.float32),
        pltpu.VMEM((2 * SEQ * DIM,), jnp.float32),
    ] + [pltpu.SemaphoreType.DMA] * (2 * NBUF + 1),
)
def _emb_kernel(ids_hbm, table_hbm, pos2_hbm, out_hbm, idx_v, buf, pos_v, *sems):
    sg = sems[:NBUF]
    so = sems[NBUF:2 * NBUF]
    si = sems[2 * NBUF]
    wid = lax.axis_index("s") * NC + lax.axis_index("c")
    base = wid * PER_W
    cbase = wid * NCH
    pltpu.sync_copy(pos2_hbm, pos_v)

    def start_ids(blk, s):
        pltpu.async_copy(ids_hbm.at[pl.ds(cbase + blk * BLK, BLK)], idx_v.at[s], si)

    def wait_ids():
        pltpu.make_async_copy(ids_hbm.at[pl.ds(cbase, BLK)], idx_v.at[0], si).wait()

    def start_gather(pc, b, s, r):
        pltpu.async_copy(table_hbm.at[idx_v.at[s, r]], buf.at[b], sg[b])

    def wait_gather(b):
        pltpu.make_async_copy(table_hbm.at[idx_v.at[0, 0]], buf.at[b], sg[b]).wait()

    def start_out(cc, b):
        pltpu.async_copy(buf.at[b], out_hbm.at[pl.ds(base + cc * K, K)], so[b])

    def wait_out(cc, b):
        pltpu.make_async_copy(
            buf.at[b], out_hbm.at[pl.ds(base + cc * K, K)], so[b]
        ).wait()

    def add_pos(cc, b):
        poff = lax.rem(cc * K, SEQ) * DIM

        def rows(r0, off):
            for rr in range(RPI):
                for j in range(DIM // LANES):
                    v = pos_v[pl.ds(off + rr * DIM + j * LANES, LANES)]
                    plsc.addupdate(
                        buf.at[b, r0 * RPI + rr, pl.ds(j * LANES, LANES)], v
                    )
            return off + RPI * DIM

        lax.fori_loop(0, K // RPI, rows, poff)

    def step(cc, i, par, wait_prev_out, start_next):
        b = i % NBUF
        b2 = (i + LEAD) % NBUF
        if wait_prev_out:
            wait_out(cc - LEAD, b2)
        if start_next:
            s = par if (i + LEAD) // BLK == 0 else 1 - par
            start_gather(cc + LEAD, b2, s, (i + LEAD) % BLK)
        wait_gather(b)
        add_pos(cc, b)
        start_out(cc, b)

    def block_body(g, par, first=False, last=False):
        c0 = g * BLK
        if not last:
            start_ids(g + 1, 1 - par)
        for i in range(BLK):
            if i == BLK - LEAD and not last:
                wait_ids()
            step(c0 + i, i, par,
                 wait_prev_out=(not first) or (i >= LEAD),
                 start_next=(not last) or (i < BLK - LEAD))

    start_ids(0, 0)
    wait_ids()
    for b in range(LEAD):
        start_gather(b, b, 0, b)

    block_body(0, 0, first=True)

    def mid(g, _):
        block_body(g, lax.rem(g, 2))
        return 0

    lax.fori_loop(1, NBLK - 1, mid, 0)

    block_body(NBLK - 1, (NBLK - 1) % 2, last=True)
    c0 = (NBLK - 1) * BLK
    for i in range(BLK - LEAD, BLK):
        wait_out(c0 + i, i % NBUF)


def kernel(input_ids, table, pos_embed):
    ids2d = input_ids.reshape(NCH * NW, K)
    pos_flat = pos_embed.reshape(SEQ * DIM)
    pos2 = jnp.concatenate([pos_flat, pos_flat])
    out = _emb_kernel(ids2d, table, pos2)
    return out.reshape(BATCH, SEQ, DIM)
